# baseline (speedup 1.0000x reference)
import jax
import jax.numpy as jnp
from jax.experimental import pallas as pl


def _mm_kernel(a_ref, w_ref, o_ref):
    k = pl.program_id(1)
    bk = a_ref.shape[1]
    w = jnp.maximum(w_ref[pl.ds(k * bk, bk), :], 0.0).astype(jnp.bfloat16)
    a = a_ref[...].astype(jnp.bfloat16)
    p = jnp.dot(a, w, preferred_element_type=jnp.float32)

    @pl.when(k == 0)
    def _():
        o_ref[...] = p

    @pl.when(k > 0)
    def _():
        o_ref[...] += p


def kernel(user_ids, item_ids, explicit_feedback, dense_weight_slice):
    M, K = explicit_feedback.shape
    N = dense_weight_slice.shape[1]
    bm, bk = 512, 1024
    return pl.pallas_call(
        _mm_kernel,
        grid=(M // bm, K // bk),
        in_specs=[
            pl.BlockSpec((bm, bk), lambda i, k: (i, k)),
            pl.BlockSpec((K, N), lambda i, k: (0, 0)),
        ],
        out_specs=pl.BlockSpec((bm, N), lambda i, k: (i, 0)),
        out_shape=jax.ShapeDtypeStruct((M, N), jnp.float32),
    )(explicit_feedback, dense_weight_slice)


# speedup vs baseline: 1.4680x; 1.4680x over previous
import jax
import jax.numpy as jnp
from jax.experimental import pallas as pl
from jax.experimental.pallas import tpu as pltpu

_BM = 256
_NBUF = 4


def _mm_kernel(a_hbm, w_ref, o_ref, buf, sems):
    M = a_hbm.shape[0]
    nblk = M // _BM

    def start(i):
        pltpu.make_async_copy(
            a_hbm.at[pl.ds(i * _BM, _BM), :], buf.at[i % _NBUF],
            sems.at[i % _NBUF],
        ).start()

    for i in range(_NBUF):
        start(i)
    w = jnp.maximum(w_ref[...], 0.0).astype(jnp.bfloat16)
    for i in range(nblk):
        pltpu.make_async_copy(
            a_hbm.at[pl.ds(i * _BM, _BM), :], buf.at[i % _NBUF],
            sems.at[i % _NBUF],
        ).wait()
        a = buf[i % _NBUF].astype(jnp.bfloat16)
        o_ref[pl.ds(i * _BM, _BM), :] = jnp.dot(
            a, w, preferred_element_type=jnp.float32)
        if i + _NBUF < nblk:
            start(i + _NBUF)


def kernel(user_ids, item_ids, explicit_feedback, dense_weight_slice):
    M, K = explicit_feedback.shape
    N = dense_weight_slice.shape[1]
    return pl.pallas_call(
        _mm_kernel,
        in_specs=[
            pl.BlockSpec(memory_space=pl.ANY),
            pl.BlockSpec((K, N), lambda: (0, 0)),
        ],
        out_specs=pl.BlockSpec((M, N), lambda: (0, 0)),
        out_shape=jax.ShapeDtypeStruct((M, N), jnp.float32),
        scratch_shapes=[
            pltpu.VMEM((_NBUF, _BM, K), jnp.float32),
            pltpu.SemaphoreType.DMA((_NBUF,)),
        ],
    )(explicit_feedback, dense_weight_slice)
